# baseline (speedup 1.0000x reference)
import functools

import jax
import jax.numpy as jnp
from jax import lax
from jax.experimental import pallas as pl
from jax.experimental.pallas import tpu as pltpu
from jax.experimental.pallas import tpu_sc as plsc

_N = 16384
_C = 16
_DIM = 64
_SPAN = 256
_OUT = _SPAN + _DIM
_NC, _NS = 2, 16
_NW = _NC * _NS
_SPW = _N // _NW
_CH = 8
_NCHUNK = _SPW // _CH
_LANES = 16
_NJ = _DIM // _LANES


def _sc_body(span_hbm, scores_hbm, cand_hbm, len_hbm, embed_hbm, out_hbm,
             idx_v, scores_v, len_v, rows_v, tail_v, gsem):
    wid = lax.axis_index("s") * _NC + lax.axis_index("c")
    row0 = wid * _SPW

    pltpu.sync_copy(cand_hbm.at[wid], idx_v)
    pltpu.sync_copy(scores_hbm.at[wid], scores_v)
    pltpu.sync_copy(len_hbm.at[wid], len_v.at[pl.ds(0, _SPW)])

    pltpu.sync_copy(span_hbm.at[wid],
                    out_hbm.at[pl.ds(row0, _SPW), pl.ds(0, _SPAN)])

    @pl.loop(0, _NCHUNK)
    def _chunk(k):
        pltpu.async_copy(embed_hbm.at[idx_v.at[k]], rows_v, gsem).wait()

        lv = len_v[pl.ds(k * _CH, _LANES)]
        maskf = jnp.where(lv > 0, 1.0, 0.0)

        for s in range(_CH):
            sv = scores_v[k * _CH + s, :]
            accs = [jnp.zeros((_LANES,), jnp.float32) for _ in range(_NJ)]
            for c in range(_C):
                w = sv[c]
                r = s * _C + c
                for j in range(_NJ):
                    accs[j] = accs[j] + w * rows_v[r, pl.ds(j * _LANES, _LANES)]
            m = maskf[s]
            for j in range(_NJ):
                tail_v[s, pl.ds(j * _LANES, _LANES)] = accs[j] * m

        pltpu.sync_copy(
            tail_v,
            out_hbm.at[pl.ds(row0 + k * _CH, _CH), pl.ds(_SPAN, _DIM)])


@functools.partial(
    pl.kernel,
    out_type=jax.ShapeDtypeStruct((_N, _OUT), jnp.float32),
    mesh=plsc.VectorSubcoreMesh(core_axis_name="c", subcore_axis_name="s"),
    compiler_params=pltpu.CompilerParams(use_tc_tiling_on_sc=False),
    scratch_types=[
        pltpu.VMEM((_NCHUNK, _CH * _C), jnp.int32),
        pltpu.VMEM((_SPW, _C), jnp.float32),
        pltpu.VMEM((_SPW + _LANES,), jnp.int32),
        pltpu.VMEM((_CH * _C, _DIM), jnp.float32),
        pltpu.VMEM((_CH, _DIM), jnp.float32),
        pltpu.SemaphoreType.DMA,
    ],
)
def _kb_kernel(span_hbm, scores_hbm, cand_hbm, len_hbm, embed_hbm, out_hbm,
               idx_v, scores_v, len_v, rows_v, tail_v, gsem):
    _sc_body(span_hbm, scores_hbm, cand_hbm, len_hbm, embed_hbm, out_hbm,
             idx_v, scores_v, len_v, rows_v, tail_v, gsem)


def kernel(span_vecs, scores, mask_candidates, embed, candidates,
           len_candidates):
    del mask_candidates
    span_r = span_vecs.reshape(_NW, _SPW, _SPAN)
    scores_r = scores.reshape(_NW, _SPW, _C)
    cand_r = candidates.reshape(_NW, _NCHUNK, _CH * _C)
    len_r = len_candidates.reshape(_NW, _SPW)
    out = _kb_kernel(span_r, scores_r, cand_r, len_r, embed)
    return out.reshape(1, _N, _OUT)
